# baseline (speedup 1.0000x reference)
import functools

import jax
import jax.numpy as jnp
from jax.experimental import pallas as pl
from jax.experimental.pallas import tpu as pltpu

B, S, D = 2, 2048, 2048
NH, DH, RD = 16, 128, 64
FP8_MAX = 448.0

M_TILE = 256


def _rope128(t, cm, sm, sp):
    return (t * cm + pltpu.roll(t, 96, axis=1) * sm
            + pltpu.roll(t, 32, axis=1) * sp)


def _fused_kernel(x_ref, wqb_ref, qn_ref, ws_ref, wk_ref, wproj_ref,
                  cos_ref, sin_ref, hadq_ref, hadk_ref, gam_ref, bet_ref,
                  q_ref, qs_ref, k_ref, ks_ref, w_ref,
                  wscr_ref, wkscr_ref, wpscr_ref, hqscr_ref, hkscr_ref):
    i = pl.program_id(0)

    @pl.when(i == 0)
    def _init():
        qn_col = qn_ref[...].reshape(D, 1)
        wscr_ref[...] = (qn_col * wqb_ref[...] * ws_ref[...]).astype(jnp.bfloat16)
        wkscr_ref[...] = wk_ref[...].astype(jnp.bfloat16)
        wpscr_ref[...] = wproj_ref[...].astype(jnp.bfloat16)
        hqscr_ref[...] = hadq_ref[...].astype(jnp.bfloat16)
        hkscr_ref[...] = hadk_ref[...].astype(jnp.bfloat16)

    c = cos_ref[:, :32]
    s = sin_ref[:, :32]
    one64 = jnp.ones((M_TILE, 64), jnp.float32)
    z32 = jnp.zeros((M_TILE, 32), jnp.float32)
    z64 = jnp.zeros((M_TILE, 64), jnp.float32)
    cm = jnp.concatenate([c, c, one64], axis=1)
    sm = jnp.concatenate([-s, z32, z64], axis=1)
    sp = jnp.concatenate([z32, s, z64], axis=1)
    xf = x_ref[...]

    rms = jax.lax.rsqrt(jnp.mean(xf * xf, axis=1, keepdims=True) + 1e-6)
    xnb = (xf * rms).astype(jnp.bfloat16)
    hadq = hqscr_ref[...]
    q = jnp.dot(xnb, wscr_ref[...], preferred_element_type=jnp.float32)
    qs_cols = []
    q_heads = []
    for h in range(NH):
        qh = _rope128(q[:, h * DH:(h + 1) * DH], cm, sm, sp).astype(jnp.bfloat16)
        qh = jnp.dot(qh, hadq, preferred_element_type=jnp.float32)
        qsc = jnp.maximum(jnp.max(jnp.abs(qh), axis=1, keepdims=True), 1e-6) / FP8_MAX
        q_heads.append(qh * (1.0 / qsc))
        qs_cols.append(qsc)
    q_ref[...] = jnp.concatenate(q_heads, axis=1).reshape(q_ref.shape)
    qs_ref[...] = jnp.concatenate(qs_cols, axis=1)

    xb = xf.astype(jnp.bfloat16)
    k = jnp.dot(xb, wkscr_ref[...], preferred_element_type=jnp.float32)
    mu = jnp.mean(k, axis=1, keepdims=True)
    kc = k - mu
    var = jnp.mean(kc * kc, axis=1, keepdims=True)
    k = kc * jax.lax.rsqrt(var + 1e-6) * gam_ref[...] + bet_ref[...]
    k = _rope128(k, cm, sm, sp).astype(jnp.bfloat16)
    k = jnp.dot(k, hkscr_ref[...], preferred_element_type=jnp.float32)
    ksc = jnp.maximum(jnp.max(jnp.abs(k), axis=1, keepdims=True), 1e-6) / FP8_MAX
    k_ref[...] = k * (1.0 / ksc)
    ks_ref[...] = ksc

    w_ref[...] = jnp.dot(xb, wpscr_ref[...],
                         preferred_element_type=jnp.float32) * (DH ** -0.5)


@functools.partial(jax.jit, static_argnames=())
def kernel(x, q_norm, q_norm_scale, w_qb, w_qb_scale, wk, w_proj, ln_gamma_k,
           ln_beta_k, cos_idx_rope, sin_idx_rope, hadamard_q, hadamard_k,
           k_cache, k_cache_scale, k_cache_index):
    del k_cache, k_cache_scale, k_cache_index
    M = B * S
    x2 = x.reshape(M, D)
    qn = (q_norm * q_norm_scale).reshape(1, D)
    ws = w_qb_scale.reshape(1, NH * DH)
    gam = ln_gamma_k.reshape(1, DH)
    bet = ln_beta_k.reshape(1, DH)

    n_tiles = M // M_TILE

    in_specs = [
            pl.BlockSpec((M_TILE, D), lambda i: (i, 0)),
            pl.BlockSpec((D, NH * DH), lambda i: (0, 0)),
            pl.BlockSpec((1, D), lambda i: (0, 0)),
            pl.BlockSpec((1, NH * DH), lambda i: (0, 0)),
            pl.BlockSpec((D, DH), lambda i: (0, 0)),
            pl.BlockSpec((D, NH), lambda i: (0, 0)),
            pl.BlockSpec((M_TILE, RD), lambda i: (i % (S // M_TILE), 0)),
            pl.BlockSpec((M_TILE, RD), lambda i: (i % (S // M_TILE), 0)),
            pl.BlockSpec((DH, DH), lambda i: (0, 0)),
            pl.BlockSpec((DH, DH), lambda i: (0, 0)),
            pl.BlockSpec((1, DH), lambda i: (0, 0)),
            pl.BlockSpec((1, DH), lambda i: (0, 0)),
    ]
    out_specs = [
            pl.BlockSpec((M_TILE, NH, DH), lambda i: (i, 0, 0)),
            pl.BlockSpec((M_TILE, NH), lambda i: (i, 0)),
            pl.BlockSpec((M_TILE, DH), lambda i: (i, 0)),
            pl.BlockSpec((M_TILE, 1), lambda i: (i, 0)),
            pl.BlockSpec((M_TILE, NH), lambda i: (i, 0)),
    ]

    q2, qs2, k2, ks2, w2 = pl.pallas_call(
        _fused_kernel,
        grid=(n_tiles,),
        in_specs=in_specs,
        out_specs=out_specs,
        out_shape=[
            jax.ShapeDtypeStruct((M, NH, DH), jnp.float32),
            jax.ShapeDtypeStruct((M, NH), jnp.float32),
            jax.ShapeDtypeStruct((M, DH), jnp.float32),
            jax.ShapeDtypeStruct((M, 1), jnp.float32),
            jax.ShapeDtypeStruct((M, NH), jnp.float32),
        ],
        scratch_shapes=[
            pltpu.VMEM((D, NH * DH), jnp.bfloat16),
            pltpu.VMEM((D, DH), jnp.bfloat16),
            pltpu.VMEM((D, NH), jnp.bfloat16),
            pltpu.VMEM((DH, DH), jnp.bfloat16),
            pltpu.VMEM((DH, DH), jnp.bfloat16),
        ],
        compiler_params=pltpu.CompilerParams(
            dimension_semantics=("arbitrary",),
        ),
    )(x2, w_qb, qn, ws, wk, w_proj, cos_idx_rope, sin_idx_rope,
      hadamard_q, hadamard_k, gam, bet)

    return (q2.reshape(B, S, NH, DH), qs2.reshape(B, S, NH),
            k2.reshape(B, S, DH), ks2.reshape(B, S), w2.reshape(B, S, NH))


# speedup vs baseline: 1.0491x; 1.0491x over previous
import functools

import jax
import jax.numpy as jnp
from jax.experimental import pallas as pl
from jax.experimental.pallas import tpu as pltpu

B, S, D = 2, 2048, 2048
NH, DH, RD = 16, 128, 64
FP8_MAX = 448.0

M_TILE = 256


def _rope128(t, cm, sm, sp):
    return (t * cm + pltpu.roll(t, 96, axis=1) * sm
            + pltpu.roll(t, 32, axis=1) * sp)


def _fused_kernel(x_ref, wqb_ref, qn_ref, ws_ref, wk_ref, wproj_ref,
                  cost_ref, sint_ref, hadq_ref, hadk_ref, gam_ref, bet_ref,
                  q_ref, qs_ref, k_ref, ks_ref, w_ref,
                  wscr_ref, wkscr_ref, wpscr_ref, hqscr_ref, hkscr_ref,
                  cscr_ref, sscr_ref):
    i = pl.program_id(0)

    @pl.when(i == 0)
    def _init():
        wscr_ref[...] = (wqb_ref[...] * ws_ref[...]).astype(jnp.bfloat16)
        wkscr_ref[...] = wk_ref[...].astype(jnp.bfloat16)
        wpscr_ref[...] = wproj_ref[...].astype(jnp.bfloat16)
        hqscr_ref[...] = hadq_ref[...].astype(jnp.bfloat16)
        hkscr_ref[...] = hadk_ref[...].astype(jnp.bfloat16)
        cscr_ref[...] = jnp.transpose(cost_ref[:32, :])
        sscr_ref[...] = jnp.transpose(sint_ref[:32, :])

    s0 = (i % (S // M_TILE)) * M_TILE
    c = cscr_ref[pl.ds(s0, M_TILE), :]
    s = sscr_ref[pl.ds(s0, M_TILE), :]
    one64 = jnp.ones((M_TILE, 64), jnp.float32)
    z32 = jnp.zeros((M_TILE, 32), jnp.float32)
    z64 = jnp.zeros((M_TILE, 64), jnp.float32)
    cm = jnp.concatenate([c, c, one64], axis=1)
    sm = jnp.concatenate([-s, z32, z64], axis=1)
    sp = jnp.concatenate([z32, s, z64], axis=1)
    xf = x_ref[...]

    rms = jax.lax.rsqrt(jnp.mean(xf * xf, axis=1, keepdims=True) + 1e-6)
    xnb = (xf * rms * qn_ref[...]).astype(jnp.bfloat16)
    hadq = hqscr_ref[...]
    q = jnp.dot(xnb, wscr_ref[...], preferred_element_type=jnp.float32)
    qs_cols = []
    q_heads = []
    for h in range(NH):
        qh = _rope128(q[:, h * DH:(h + 1) * DH], cm, sm, sp).astype(jnp.bfloat16)
        qh = jnp.dot(qh, hadq, preferred_element_type=jnp.float32)
        qsc = jnp.maximum(jnp.max(jnp.abs(qh), axis=1, keepdims=True), 1e-6) / FP8_MAX
        q_heads.append(qh * (1.0 / qsc))
        qs_cols.append(qsc)
    q_ref[...] = jnp.concatenate(q_heads, axis=1).reshape(q_ref.shape)
    qs_ref[...] = jnp.transpose(jnp.concatenate(qs_cols, axis=1))

    xb = xf.astype(jnp.bfloat16)
    k = jnp.dot(xb, wkscr_ref[...], preferred_element_type=jnp.float32)
    mu = jnp.mean(k, axis=1, keepdims=True)
    kc = k - mu
    var = jnp.mean(kc * kc, axis=1, keepdims=True)
    k = kc * jax.lax.rsqrt(var + 1e-6) * gam_ref[...] + bet_ref[...]
    k = _rope128(k, cm, sm, sp).astype(jnp.bfloat16)
    k = jnp.dot(k, hkscr_ref[...], preferred_element_type=jnp.float32)
    ksc = jnp.maximum(jnp.max(jnp.abs(k), axis=1, keepdims=True), 1e-6) / FP8_MAX
    k_ref[...] = k * (1.0 / ksc)
    ks_ref[...] = jnp.transpose(ksc).reshape(1, 1, M_TILE)

    w_ref[...] = jnp.transpose(
        jnp.dot(xb, wpscr_ref[...],
                preferred_element_type=jnp.float32) * (DH ** -0.5))


@functools.partial(jax.jit, static_argnames=())
def kernel(x, q_norm, q_norm_scale, w_qb, w_qb_scale, wk, w_proj, ln_gamma_k,
           ln_beta_k, cos_idx_rope, sin_idx_rope, hadamard_q, hadamard_k,
           k_cache, k_cache_scale, k_cache_index):
    del k_cache, k_cache_scale, k_cache_index
    M = B * S
    x2 = x.reshape(M, D)
    qn = (q_norm * q_norm_scale).reshape(1, D)
    ws = w_qb_scale.reshape(1, NH * DH)
    cost = cos_idx_rope.T
    sint = sin_idx_rope.T
    gam = ln_gamma_k.reshape(1, DH)
    bet = ln_beta_k.reshape(1, DH)

    n_tiles = M // M_TILE

    in_specs = [
            pl.BlockSpec((M_TILE, D), lambda i: (i, 0)),
            pl.BlockSpec((D, NH * DH), lambda i: (0, 0)),
            pl.BlockSpec((1, D), lambda i: (0, 0)),
            pl.BlockSpec((1, NH * DH), lambda i: (0, 0)),
            pl.BlockSpec((D, DH), lambda i: (0, 0)),
            pl.BlockSpec((D, NH), lambda i: (0, 0)),
            pl.BlockSpec((RD, S), lambda i: (0, 0)),
            pl.BlockSpec((RD, S), lambda i: (0, 0)),
            pl.BlockSpec((DH, DH), lambda i: (0, 0)),
            pl.BlockSpec((DH, DH), lambda i: (0, 0)),
            pl.BlockSpec((1, DH), lambda i: (0, 0)),
            pl.BlockSpec((1, DH), lambda i: (0, 0)),
    ]
    out_specs = [
            pl.BlockSpec((M_TILE, NH, DH), lambda i: (i, 0, 0)),
            pl.BlockSpec((NH, M_TILE), lambda i: (jax.lax.div(i, jnp.int32(S // M_TILE)), jax.lax.rem(i, jnp.int32(S // M_TILE)))),
            pl.BlockSpec((M_TILE, DH), lambda i: (i, 0)),
            pl.BlockSpec((1, 1, M_TILE), lambda i: (jax.lax.div(i, jnp.int32(S // M_TILE)), 0, jax.lax.rem(i, jnp.int32(S // M_TILE)))),
            pl.BlockSpec((NH, M_TILE), lambda i: (jax.lax.div(i, jnp.int32(S // M_TILE)), jax.lax.rem(i, jnp.int32(S // M_TILE)))),
    ]

    q2, qs2, k2, ks2, w2 = pl.pallas_call(
        _fused_kernel,
        grid=(n_tiles,),
        in_specs=in_specs,
        out_specs=out_specs,
        out_shape=[
            jax.ShapeDtypeStruct((M, NH, DH), jnp.float32),
            jax.ShapeDtypeStruct((B * NH, S), jnp.float32),
            jax.ShapeDtypeStruct((M, DH), jnp.float32),
            jax.ShapeDtypeStruct((B, 1, S), jnp.float32),
            jax.ShapeDtypeStruct((B * NH, S), jnp.float32),
        ],
        scratch_shapes=[
            pltpu.VMEM((D, NH * DH), jnp.bfloat16),
            pltpu.VMEM((D, DH), jnp.bfloat16),
            pltpu.VMEM((D, NH), jnp.bfloat16),
            pltpu.VMEM((DH, DH), jnp.bfloat16),
            pltpu.VMEM((DH, DH), jnp.bfloat16),
            pltpu.VMEM((S, 32), jnp.float32),
            pltpu.VMEM((S, 32), jnp.float32),
        ],
        compiler_params=pltpu.CompilerParams(
            dimension_semantics=("arbitrary",),
        ),
    )(x2, w_qb, qn, ws, wk, w_proj, cost, sint,
      hadamard_q, hadamard_k, gam, bet)

    return (q2.reshape(B, S, NH, DH),
            qs2.reshape(B, NH, S).transpose(0, 2, 1),
            k2.reshape(B, S, DH), ks2.reshape(B, S),
            w2.reshape(B, NH, S).transpose(0, 2, 1))
